# baseline (speedup 1.0000x reference)
import functools

import jax
import jax.numpy as jnp
from jax import lax
from jax.experimental import pallas as pl
from jax.experimental.pallas import tpu as pltpu
from jax.experimental.pallas import tpu_sc as plsc

N = 10000
E0 = 320000
ET = E0 + N
NODE_DIM = 48
HD = 128
EDGE_DIM = 32
GLOBAL_DIM = 32
EPG = 32000
NPG = 1000
CNT = float(EPG + NPG)

NW = 32
CHUNK = 128
CPW = 88
NCH = NW * CPW
EPAD = NCH * CHUNK

BLK = 3072
NBLK = -(-ET // BLK)

_EPS = 1e-5


def _ln_rows(v, g, b):
    mu = jnp.mean(v, axis=-1, keepdims=True)
    var = jnp.mean(v * v, axis=-1, keepdims=True) - mu * mu
    return (v - mu) * jax.lax.rsqrt(var + _EPS) * g + b


def _node_body(x_ref, aid_ref, sid_ref, aemb_ref, semb_ref,
               lncg_ref, lncb_ref, wp1_ref, bp1_ref, wp2_ref, bp2_ref,
               wd_ref, bd_ref, lnng_ref, lnnb_ref, h_ref):
    x = x_ref[...]
    h = _ln_rows(x, lncg_ref[...], lncb_ref[...])
    h = jnp.maximum(jnp.dot(h, wp1_ref[...], preferred_element_type=jnp.float32) + bp1_ref[...], 0.0)
    h = jnp.maximum(jnp.dot(h, wp2_ref[...], preferred_element_type=jnp.float32) + bp2_ref[...], 0.0)
    h = jnp.maximum(jnp.dot(h, wd_ref[...], preferred_element_type=jnp.float32) + bd_ref[...], 0.0)
    rows = x.shape[0]
    aid = aid_ref[...]
    sid = sid_ref[...]
    na = aemb_ref.shape[0]
    ns = semb_ref.shape[0]
    aoh = (aid == lax.broadcasted_iota(jnp.int32, (rows, na), 1)).astype(jnp.float32)
    soh = (sid == lax.broadcasted_iota(jnp.int32, (rows, ns), 1)).astype(jnp.float32)
    a_e = jnp.dot(aoh, aemb_ref[...], preferred_element_type=jnp.float32)
    s_e = jnp.dot(soh, semb_ref[...], preferred_element_type=jnp.float32)
    hn = _ln_rows(h + a_e + s_e, lnng_ref[...], lnnb_ref[...])
    h_ref[...] = jnp.concatenate(
        [hn, jnp.zeros((rows, HD - NODE_DIM), jnp.float32)], axis=1)


def _node_init(x, atom_ids, aa_ids, atom_emb, aa_emb,
               ln_c_g, ln_c_b, W_p1, b_p1, W_p2, b_p2, W_d, b_d, ln_n_g, ln_n_b):
    nb = 10
    rows = N // nb
    full = lambda s: pl.BlockSpec(s, lambda b: (0, 0))
    return pl.pallas_call(
        _node_body,
        grid=(nb,),
        in_specs=[
            pl.BlockSpec((rows, 3), lambda b: (b, 0)),
            pl.BlockSpec((rows, 1), lambda b: (b, 0)),
            pl.BlockSpec((rows, 1), lambda b: (b, 0)),
            full(atom_emb.shape), full(aa_emb.shape),
            full((1, 3)), full((1, 3)),
            full(W_p1.shape), full((1, 12)),
            full(W_p2.shape), full((1, 48)),
            full(W_d.shape), full((1, 48)),
            full((1, 48)), full((1, 48)),
        ],
        out_specs=pl.BlockSpec((rows, HD), lambda b: (b, 0)),
        out_shape=jax.ShapeDtypeStruct((N, HD), jnp.float32),
    )(x, atom_ids.reshape(N, 1).astype(jnp.int32), aa_ids.reshape(N, 1).astype(jnp.int32),
      atom_emb, aa_emb,
      ln_c_g.reshape(1, 3), ln_c_b.reshape(1, 3),
      W_p1, b_p1.reshape(1, 12), W_p2, b_p2.reshape(1, 48),
      W_d, b_d.reshape(1, 48), ln_n_g.reshape(1, 48), ln_n_b.reshape(1, 48))


def _sc_gather(h, idx0, idx1):
    mesh = plsc.VectorSubcoreMesh(core_axis_name="c", subcore_axis_name="s")

    @functools.partial(
        pl.kernel,
        out_type=[jax.ShapeDtypeStruct((EPAD, HD), jnp.float32),
                  jax.ShapeDtypeStruct((EPAD, HD), jnp.float32)],
        mesh=mesh,
        scratch_types=[
            pltpu.VMEM((CPW, CHUNK), jnp.int32),
            pltpu.VMEM((CPW, CHUNK), jnp.int32),
            pltpu.VMEM((CHUNK, HD), jnp.float32),
            pltpu.VMEM((CHUNK, HD), jnp.float32),
            pltpu.SemaphoreType.DMA,
            pltpu.SemaphoreType.DMA,
        ],
    )
    def k(h_hbm, i0_hbm, i1_hbm, hr_hbm, hc_hbm, i0v, i1v, b0, b1, s0, s1):
        wid = lax.axis_index("s") * 2 + lax.axis_index("c")
        cbase = wid * CPW
        pltpu.sync_copy(i0_hbm.at[wid], i0v)
        pltpu.sync_copy(i1_hbm.at[wid], i1v)

        def body(j, carry):
            c0 = pltpu.async_copy(h_hbm.at[i0v.at[j]], b0, s0)
            c1 = pltpu.async_copy(h_hbm.at[i1v.at[j]], b1, s1)
            dst = pl.ds((cbase + j) * CHUNK, CHUNK)
            c0.wait()
            pltpu.sync_copy(b0, hr_hbm.at[dst, :])
            c1.wait()
            pltpu.sync_copy(b1, hc_hbm.at[dst, :])
            return carry

        lax.fori_loop(0, CPW, body, 0, unroll=False)

    return k(h, idx0, idx1)


def _edge_attr_blk(hr, hc, we, be, lneg, lneb):
    her = jnp.maximum(jnp.dot(hr, we, preferred_element_type=jnp.float32) + be, 0.0)
    hec = jnp.maximum(jnp.dot(hc, we, preferred_element_type=jnp.float32) + be, 0.0)
    return _ln_rows((her + hec) * 0.5, lneg, lneb)


def _rowids(b, axis):
    if axis == 0:
        return lax.broadcasted_iota(jnp.int32, (BLK, 16), 0) + b * BLK
    return lax.broadcasted_iota(jnp.int32, (16, BLK), 1) + b * BLK


def _gid(rid):
    g = jnp.where(rid < E0, rid // EPG, (rid - E0) // NPG)
    return jnp.where(rid < ET, g, -1)


def _pass1_body(hr_ref, hc_ref, we_ref, be_ref, lneg_ref, lneb_ref,
                wg_ref, bg_ref, psum_ref):
    b = pl.program_id(0)
    ea = _edge_attr_blk(hr_ref[...], hc_ref[...], we_ref[...], be_ref[...],
                        lneg_ref[...], lneb_ref[...])
    eg = jnp.maximum(jnp.dot(ea, wg_ref[...], preferred_element_type=jnp.float32) + bg_ref[...], 0.0)
    ohT = (_gid(_rowids(b, 1)) == lax.broadcasted_iota(jnp.int32, (16, BLK), 0)
           ).astype(jnp.float32)
    part = jnp.dot(ohT, eg, preferred_element_type=jnp.float32)

    @pl.when(b == 0)
    def _():
        psum_ref[...] = jnp.zeros_like(psum_ref)

    psum_ref[...] += part


def _pass1(hr, hc, W_e128, b_e, ln_e_g, ln_e_b, W_g, b_g):
    full = lambda s: pl.BlockSpec(s, lambda b: (0, 0))
    return pl.pallas_call(
        _pass1_body,
        grid=(NBLK,),
        in_specs=[
            pl.BlockSpec((BLK, HD), lambda b: (b, 0)),
            pl.BlockSpec((BLK, HD), lambda b: (b, 0)),
            full((HD, EDGE_DIM)), full((1, EDGE_DIM)),
            full((1, EDGE_DIM)), full((1, EDGE_DIM)),
            full(W_g.shape), full((1, GLOBAL_DIM)),
        ],
        out_specs=pl.BlockSpec((16, GLOBAL_DIM), lambda b: (0, 0)),
        out_shape=jax.ShapeDtypeStruct((16, GLOBAL_DIM), jnp.float32),
    )(hr, hc, W_e128, b_e.reshape(1, EDGE_DIM), ln_e_g.reshape(1, EDGE_DIM),
      ln_e_b.reshape(1, EDGE_DIM), W_g, b_g.reshape(1, GLOBAL_DIM))


def _pass2_body(hr_ref, hc_ref, ps_ref, we_ref, be_ref, lneg_ref, lneb_ref,
                lnug_ref, lnub_ref, wa_ref, wb_ref, wc_ref, wd_ref, bm1_ref,
                wm2_ref, bm2_ref, lnog_ref, lnob_ref, out_ref):
    b = pl.program_id(0)
    u = _ln_rows(ps_ref[...] * (1.0 / CNT), lnug_ref[...], lnub_ref[...])
    oh = (_gid(_rowids(b, 0)) == lax.broadcasted_iota(jnp.int32, (BLK, 16), 1)
          ).astype(jnp.float32)
    usel = jnp.dot(oh, u, preferred_element_type=jnp.float32)

    hr = hr_ref[...]
    hc = hc_ref[...]
    ea = _edge_attr_blk(hr, hc, we_ref[...], be_ref[...], lneg_ref[...], lneb_ref[...])
    t = (jnp.dot(hr, wa_ref[...], preferred_element_type=jnp.float32)
         + jnp.dot(hc, wb_ref[...], preferred_element_type=jnp.float32)
         + jnp.dot(ea, wc_ref[...], preferred_element_type=jnp.float32)
         + jnp.dot(usel, wd_ref[...], preferred_element_type=jnp.float32)
         + bm1_ref[...])
    t = jnp.maximum(t, 0.0)
    o = jnp.dot(t, wm2_ref[...], preferred_element_type=jnp.float32) + bm2_ref[...] + ea
    out_ref[...] = _ln_rows(o, lnog_ref[...], lnob_ref[...]).T


def _pass2(hr, hc, psums, W_e128, b_e, ln_e_g, ln_e_b, ln_u_g, ln_u_b,
           wa, wb, wc, wd, b_m1, W_m2, b_m2, ln_o_g, ln_o_b):
    full = lambda s: pl.BlockSpec(s, lambda b: (0, 0))
    h1 = wa.shape[1]
    return pl.pallas_call(
        _pass2_body,
        grid=(NBLK,),
        in_specs=[
            pl.BlockSpec((BLK, HD), lambda b: (b, 0)),
            pl.BlockSpec((BLK, HD), lambda b: (b, 0)),
            full((16, GLOBAL_DIM)),
            full((HD, EDGE_DIM)), full((1, EDGE_DIM)),
            full((1, EDGE_DIM)), full((1, EDGE_DIM)),
            full((1, GLOBAL_DIM)), full((1, GLOBAL_DIM)),
            full((HD, h1)), full((HD, h1)),
            full((EDGE_DIM, h1)), full((GLOBAL_DIM, h1)),
            full((1, h1)),
            full(W_m2.shape), full((1, EDGE_DIM)),
            full((1, EDGE_DIM)), full((1, EDGE_DIM)),
        ],
        out_specs=pl.BlockSpec((EDGE_DIM, BLK), lambda b: (0, b)),
        out_shape=jax.ShapeDtypeStruct((EDGE_DIM, ET), jnp.float32),
    )(hr, hc, psums, W_e128, b_e.reshape(1, EDGE_DIM), ln_e_g.reshape(1, EDGE_DIM),
      ln_e_b.reshape(1, EDGE_DIM), ln_u_g.reshape(1, GLOBAL_DIM),
      ln_u_b.reshape(1, GLOBAL_DIM), wa, wb, wc, wd, b_m1.reshape(1, h1),
      W_m2, b_m2.reshape(1, EDGE_DIM), ln_o_g.reshape(1, EDGE_DIM),
      ln_o_b.reshape(1, EDGE_DIM))


def kernel(x, atom_ids, aa_ids, edge_index, ln_c_g, ln_c_b, W_p1, b_p1, W_p2, b_p2,
           W_d, b_d, atom_emb, aa_emb, ln_n_g, ln_n_b, W_e, b_e, ln_e_g, ln_e_b,
           W_g, b_g, ln_u_g, ln_u_b, W_m1, b_m1, W_m2, b_m2, ln_o_g, ln_o_b):
    h = _node_init(x, atom_ids, aa_ids, atom_emb, aa_emb,
                   ln_c_g, ln_c_b, W_p1, b_p1, W_p2, b_p2, W_d, b_d, ln_n_g, ln_n_b)

    loops = jnp.arange(N, dtype=jnp.int32)
    pad = jnp.zeros((EPAD - ET,), jnp.int32)
    ei0 = jnp.concatenate([edge_index[0].astype(jnp.int32), loops, pad]).reshape(NW, CPW, CHUNK)
    ei1 = jnp.concatenate([edge_index[1].astype(jnp.int32), loops, pad]).reshape(NW, CPW, CHUNK)

    hr, hc = _sc_gather(h, ei0, ei1)

    zpad = jnp.zeros((HD - NODE_DIM,), jnp.float32)
    W_e128 = jnp.concatenate([W_e, jnp.zeros((HD - NODE_DIM, EDGE_DIM), jnp.float32)])
    h1 = W_m1.shape[1]
    zw = jnp.zeros((HD - NODE_DIM, h1), jnp.float32)
    wa = jnp.concatenate([W_m1[:NODE_DIM], zw])
    wb = jnp.concatenate([W_m1[NODE_DIM:2 * NODE_DIM], zw])
    wc = W_m1[2 * NODE_DIM:2 * NODE_DIM + EDGE_DIM]
    wd = W_m1[2 * NODE_DIM + EDGE_DIM:]
    del zpad

    psums = _pass1(hr, hc, W_e128, b_e, ln_e_g, ln_e_b, W_g, b_g)
    out_t = _pass2(hr, hc, psums, W_e128, b_e, ln_e_g, ln_e_b, ln_u_g, ln_u_b,
                   wa, wb, wc, wd, b_m1, W_m2, b_m2, ln_o_g, ln_o_b)
    return out_t.T


# speedup vs baseline: 1.0011x; 1.0011x over previous
import functools

import jax
import jax.numpy as jnp
from jax import lax
from jax.experimental import pallas as pl
from jax.experimental.pallas import tpu as pltpu
from jax.experimental.pallas import tpu_sc as plsc

N = 10000
E0 = 320000
ET = E0 + N
NODE_DIM = 48
HD = 128
EDGE_DIM = 32
GLOBAL_DIM = 32
EPG = 32000
NPG = 1000
CNT = float(EPG + NPG)

NW = 32
CHUNK = 128
CPW = 88
NCH = NW * CPW
EPAD = NCH * CHUNK

BLK = 3072
NBLK = -(-ET // BLK)

_EPS = 1e-5


def _ln_rows(v, g, b):
    mu = jnp.mean(v, axis=-1, keepdims=True)
    var = jnp.mean(v * v, axis=-1, keepdims=True) - mu * mu
    return (v - mu) * jax.lax.rsqrt(var + _EPS) * g + b


def _node_body(x_ref, aid_ref, sid_ref, aemb_ref, semb_ref,
               lncg_ref, lncb_ref, wp1_ref, bp1_ref, wp2_ref, bp2_ref,
               wd_ref, bd_ref, lnng_ref, lnnb_ref, h_ref):
    x = x_ref[...]
    h = _ln_rows(x, lncg_ref[...], lncb_ref[...])
    h = jnp.maximum(jnp.dot(h, wp1_ref[...], preferred_element_type=jnp.float32) + bp1_ref[...], 0.0)
    h = jnp.maximum(jnp.dot(h, wp2_ref[...], preferred_element_type=jnp.float32) + bp2_ref[...], 0.0)
    h = jnp.maximum(jnp.dot(h, wd_ref[...], preferred_element_type=jnp.float32) + bd_ref[...], 0.0)
    rows = x.shape[0]
    aid = aid_ref[...]
    sid = sid_ref[...]
    na = aemb_ref.shape[0]
    ns = semb_ref.shape[0]
    aoh = (aid == lax.broadcasted_iota(jnp.int32, (rows, na), 1)).astype(jnp.float32)
    soh = (sid == lax.broadcasted_iota(jnp.int32, (rows, ns), 1)).astype(jnp.float32)
    a_e = jnp.dot(aoh, aemb_ref[...], preferred_element_type=jnp.float32)
    s_e = jnp.dot(soh, semb_ref[...], preferred_element_type=jnp.float32)
    hn = _ln_rows(h + a_e + s_e, lnng_ref[...], lnnb_ref[...])
    h_ref[...] = jnp.concatenate(
        [hn, jnp.zeros((rows, HD - NODE_DIM), jnp.float32)], axis=1)


def _node_init(x, atom_ids, aa_ids, atom_emb, aa_emb,
               ln_c_g, ln_c_b, W_p1, b_p1, W_p2, b_p2, W_d, b_d, ln_n_g, ln_n_b):
    nb = 10
    rows = N // nb
    full = lambda s: pl.BlockSpec(s, lambda b: (0, 0))
    return pl.pallas_call(
        _node_body,
        grid=(nb,),
        in_specs=[
            pl.BlockSpec((rows, 3), lambda b: (b, 0)),
            pl.BlockSpec((rows, 1), lambda b: (b, 0)),
            pl.BlockSpec((rows, 1), lambda b: (b, 0)),
            full(atom_emb.shape), full(aa_emb.shape),
            full((1, 3)), full((1, 3)),
            full(W_p1.shape), full((1, 12)),
            full(W_p2.shape), full((1, 48)),
            full(W_d.shape), full((1, 48)),
            full((1, 48)), full((1, 48)),
        ],
        out_specs=pl.BlockSpec((rows, HD), lambda b: (b, 0)),
        out_shape=jax.ShapeDtypeStruct((N, HD), jnp.float32),
    )(x, atom_ids.reshape(N, 1).astype(jnp.int32), aa_ids.reshape(N, 1).astype(jnp.int32),
      atom_emb, aa_emb,
      ln_c_g.reshape(1, 3), ln_c_b.reshape(1, 3),
      W_p1, b_p1.reshape(1, 12), W_p2, b_p2.reshape(1, 48),
      W_d, b_d.reshape(1, 48), ln_n_g.reshape(1, 48), ln_n_b.reshape(1, 48))


def _sc_gather(h, idx0, idx1):
    mesh = plsc.VectorSubcoreMesh(core_axis_name="c", subcore_axis_name="s")

    @functools.partial(
        pl.kernel,
        out_type=[jax.ShapeDtypeStruct((EPAD, HD), jnp.float32),
                  jax.ShapeDtypeStruct((EPAD, HD), jnp.float32)],
        mesh=mesh,
        scratch_types=[
            pltpu.VMEM((CPW, CHUNK), jnp.int32),
            pltpu.VMEM((CPW, CHUNK), jnp.int32),
            pltpu.VMEM((CHUNK, HD), jnp.float32),
            pltpu.VMEM((CHUNK, HD), jnp.float32),
            pltpu.VMEM((CHUNK, HD), jnp.float32),
            pltpu.VMEM((CHUNK, HD), jnp.float32),
            pltpu.SemaphoreType.DMA,
            pltpu.SemaphoreType.DMA,
            pltpu.SemaphoreType.DMA,
            pltpu.SemaphoreType.DMA,
        ],
        compiler_params=pltpu.CompilerParams(use_tc_tiling_on_sc=False),
    )
    def k(h_hbm, i0_hbm, i1_hbm, hr_hbm, hc_hbm, i0v, i1v,
          ar, ac, br, bc, sar, sac, sbr, sbc):
        wid = lax.axis_index("s") * 2 + lax.axis_index("c")
        cbase = wid * CPW
        pltpu.sync_copy(i0_hbm.at[wid], i0v)
        pltpu.sync_copy(i1_hbm.at[wid], i1v)

        def gather(j, bufr, bufc, semr, semc):
            pltpu.async_copy(h_hbm.at[i0v.at[j]], bufr, semr)
            pltpu.async_copy(h_hbm.at[i1v.at[j]], bufc, semc)

        def wait(bufr, bufc, semr, semc):
            pltpu.make_async_copy(h_hbm.at[i0v.at[0]], bufr, semr).wait()
            pltpu.make_async_copy(h_hbm.at[i1v.at[0]], bufc, semc).wait()

        def write(j, bufr, bufc):
            dst = pl.ds((cbase + j) * CHUNK, CHUNK)
            pltpu.sync_copy(bufr, hr_hbm.at[dst, :])
            pltpu.sync_copy(bufc, hc_hbm.at[dst, :])

        gather(0, ar, ac, sar, sac)

        def body(i2, carry):
            j = i2 * 2
            gather(j + 1, br, bc, sbr, sbc)
            wait(ar, ac, sar, sac)
            write(j, ar, ac)
            gather(j + 2, ar, ac, sar, sac)
            wait(br, bc, sbr, sbc)
            write(j + 1, br, bc)
            return carry

        lax.fori_loop(0, (CPW - 2) // 2, body, 0, unroll=False)
        j = CPW - 2
        gather(j + 1, br, bc, sbr, sbc)
        wait(ar, ac, sar, sac)
        write(j, ar, ac)
        wait(br, bc, sbr, sbc)
        write(j + 1, br, bc)

    return k(h, idx0, idx1)


def _edge_attr_blk(hr, hc, we, be, lneg, lneb):
    her = jnp.maximum(jnp.dot(hr, we, preferred_element_type=jnp.float32) + be, 0.0)
    hec = jnp.maximum(jnp.dot(hc, we, preferred_element_type=jnp.float32) + be, 0.0)
    return _ln_rows((her + hec) * 0.5, lneg, lneb)


def _rowids(b, axis):
    if axis == 0:
        return lax.broadcasted_iota(jnp.int32, (BLK, 16), 0) + b * BLK
    return lax.broadcasted_iota(jnp.int32, (16, BLK), 1) + b * BLK


def _gid(rid):
    g = jnp.where(rid < E0, rid // EPG, (rid - E0) // NPG)
    return jnp.where(rid < ET, g, -1)


def _pass1_body(hr_ref, hc_ref, we_ref, be_ref, lneg_ref, lneb_ref,
                wg_ref, bg_ref, psum_ref):
    b = pl.program_id(0)
    ea = _edge_attr_blk(hr_ref[:, :NODE_DIM], hc_ref[:, :NODE_DIM],
                        we_ref[...], be_ref[...],
                        lneg_ref[...], lneb_ref[...])
    eg = jnp.maximum(jnp.dot(ea, wg_ref[...], preferred_element_type=jnp.float32) + bg_ref[...], 0.0)
    ohT = (_gid(_rowids(b, 1)) == lax.broadcasted_iota(jnp.int32, (16, BLK), 0)
           ).astype(jnp.float32)
    part = jnp.dot(ohT, eg, preferred_element_type=jnp.float32)

    @pl.when(b == 0)
    def _():
        psum_ref[...] = jnp.zeros_like(psum_ref)

    psum_ref[...] += part


def _pass1(hr, hc, W_e128, b_e, ln_e_g, ln_e_b, W_g, b_g):
    full = lambda s: pl.BlockSpec(s, lambda b: (0, 0))
    return pl.pallas_call(
        _pass1_body,
        grid=(NBLK,),
        in_specs=[
            pl.BlockSpec((BLK, HD), lambda b: (b, 0)),
            pl.BlockSpec((BLK, HD), lambda b: (b, 0)),
            full((NODE_DIM, EDGE_DIM)), full((1, EDGE_DIM)),
            full((1, EDGE_DIM)), full((1, EDGE_DIM)),
            full(W_g.shape), full((1, GLOBAL_DIM)),
        ],
        out_specs=pl.BlockSpec((16, GLOBAL_DIM), lambda b: (0, 0)),
        out_shape=jax.ShapeDtypeStruct((16, GLOBAL_DIM), jnp.float32),
    )(hr, hc, W_e128, b_e.reshape(1, EDGE_DIM), ln_e_g.reshape(1, EDGE_DIM),
      ln_e_b.reshape(1, EDGE_DIM), W_g, b_g.reshape(1, GLOBAL_DIM))


def _pass2_body(hr_ref, hc_ref, ps_ref, we_ref, be_ref, lneg_ref, lneb_ref,
                lnug_ref, lnub_ref, wa_ref, wb_ref, wc_ref, wd_ref, bm1_ref,
                wm2_ref, bm2_ref, lnog_ref, lnob_ref, out_ref):
    b = pl.program_id(0)
    u = _ln_rows(ps_ref[...] * (1.0 / CNT), lnug_ref[...], lnub_ref[...])
    oh = (_gid(_rowids(b, 0)) == lax.broadcasted_iota(jnp.int32, (BLK, 16), 1)
          ).astype(jnp.float32)
    usel = jnp.dot(oh, u, preferred_element_type=jnp.float32)

    hr = hr_ref[:, :NODE_DIM]
    hc = hc_ref[:, :NODE_DIM]
    ea = _edge_attr_blk(hr, hc, we_ref[...], be_ref[...], lneg_ref[...], lneb_ref[...])
    t = (jnp.dot(hr, wa_ref[...], preferred_element_type=jnp.float32)
         + jnp.dot(hc, wb_ref[...], preferred_element_type=jnp.float32)
         + jnp.dot(ea, wc_ref[...], preferred_element_type=jnp.float32)
         + jnp.dot(usel, wd_ref[...], preferred_element_type=jnp.float32)
         + bm1_ref[...])
    t = jnp.maximum(t, 0.0)
    o = jnp.dot(t, wm2_ref[...], preferred_element_type=jnp.float32) + bm2_ref[...] + ea
    out_ref[...] = _ln_rows(o, lnog_ref[...], lnob_ref[...]).T


def _pass2(hr, hc, psums, W_e128, b_e, ln_e_g, ln_e_b, ln_u_g, ln_u_b,
           wa, wb, wc, wd, b_m1, W_m2, b_m2, ln_o_g, ln_o_b):
    full = lambda s: pl.BlockSpec(s, lambda b: (0, 0))
    h1 = wa.shape[1]
    return pl.pallas_call(
        _pass2_body,
        grid=(NBLK,),
        in_specs=[
            pl.BlockSpec((BLK, HD), lambda b: (b, 0)),
            pl.BlockSpec((BLK, HD), lambda b: (b, 0)),
            full((16, GLOBAL_DIM)),
            full((NODE_DIM, EDGE_DIM)), full((1, EDGE_DIM)),
            full((1, EDGE_DIM)), full((1, EDGE_DIM)),
            full((1, GLOBAL_DIM)), full((1, GLOBAL_DIM)),
            full((NODE_DIM, h1)), full((NODE_DIM, h1)),
            full((EDGE_DIM, h1)), full((GLOBAL_DIM, h1)),
            full((1, h1)),
            full(W_m2.shape), full((1, EDGE_DIM)),
            full((1, EDGE_DIM)), full((1, EDGE_DIM)),
        ],
        out_specs=pl.BlockSpec((EDGE_DIM, BLK), lambda b: (0, b)),
        out_shape=jax.ShapeDtypeStruct((EDGE_DIM, ET), jnp.float32),
    )(hr, hc, psums, W_e128, b_e.reshape(1, EDGE_DIM), ln_e_g.reshape(1, EDGE_DIM),
      ln_e_b.reshape(1, EDGE_DIM), ln_u_g.reshape(1, GLOBAL_DIM),
      ln_u_b.reshape(1, GLOBAL_DIM), wa, wb, wc, wd, b_m1.reshape(1, h1),
      W_m2, b_m2.reshape(1, EDGE_DIM), ln_o_g.reshape(1, EDGE_DIM),
      ln_o_b.reshape(1, EDGE_DIM))


def kernel(x, atom_ids, aa_ids, edge_index, ln_c_g, ln_c_b, W_p1, b_p1, W_p2, b_p2,
           W_d, b_d, atom_emb, aa_emb, ln_n_g, ln_n_b, W_e, b_e, ln_e_g, ln_e_b,
           W_g, b_g, ln_u_g, ln_u_b, W_m1, b_m1, W_m2, b_m2, ln_o_g, ln_o_b):
    h = _node_init(x, atom_ids, aa_ids, atom_emb, aa_emb,
                   ln_c_g, ln_c_b, W_p1, b_p1, W_p2, b_p2, W_d, b_d, ln_n_g, ln_n_b)

    loops = jnp.arange(N, dtype=jnp.int32)
    pad = jnp.zeros((EPAD - ET,), jnp.int32)
    ei0 = jnp.concatenate([edge_index[0].astype(jnp.int32), loops, pad]).reshape(NW, CPW, CHUNK)
    ei1 = jnp.concatenate([edge_index[1].astype(jnp.int32), loops, pad]).reshape(NW, CPW, CHUNK)

    hr, hc = _sc_gather(h, ei0, ei1)

    wa = W_m1[:NODE_DIM]
    wb = W_m1[NODE_DIM:2 * NODE_DIM]
    wc = W_m1[2 * NODE_DIM:2 * NODE_DIM + EDGE_DIM]
    wd = W_m1[2 * NODE_DIM + EDGE_DIM:]

    psums = _pass1(hr, hc, W_e, b_e, ln_e_g, ln_e_b, W_g, b_g)
    out_t = _pass2(hr, hc, psums, W_e, b_e, ln_e_g, ln_e_b, ln_u_g, ln_u_b,
                   wa, wb, wc, wd, b_m1, W_m2, b_m2, ln_o_g, ln_o_b)
    return out_t.T


# speedup vs baseline: 2.1042x; 2.1018x over previous
import functools

import jax
import jax.numpy as jnp
from jax import lax
from jax.experimental import pallas as pl
from jax.experimental.pallas import tpu as pltpu
from jax.experimental.pallas import tpu_sc as plsc

N = 10000
E0 = 320000
ET = E0 + N
NODE_DIM = 48
HD = 128
EDGE_DIM = 32
GLOBAL_DIM = 32
EPG = 32000
NPG = 1000
CNT = float(EPG + NPG)

NW = 32
CHUNK = 128
CPW = 88
NCH = NW * CPW
EPAD = NCH * CHUNK

BLK = 3072
NBLK = -(-ET // BLK)

_EPS = 1e-5


def _ln_rows(v, g, b):
    mu = jnp.mean(v, axis=-1, keepdims=True)
    var = jnp.mean(v * v, axis=-1, keepdims=True) - mu * mu
    return (v - mu) * jax.lax.rsqrt(var + _EPS) * g + b


def _node_body(x_ref, aid_ref, sid_ref, aemb_ref, semb_ref,
               lncg_ref, lncb_ref, wp1_ref, bp1_ref, wp2_ref, bp2_ref,
               wd_ref, bd_ref, lnng_ref, lnnb_ref, h_ref):
    x = x_ref[...]
    h = _ln_rows(x, lncg_ref[...], lncb_ref[...])
    h = jnp.maximum(jnp.dot(h, wp1_ref[...], preferred_element_type=jnp.float32) + bp1_ref[...], 0.0)
    h = jnp.maximum(jnp.dot(h, wp2_ref[...], preferred_element_type=jnp.float32) + bp2_ref[...], 0.0)
    h = jnp.maximum(jnp.dot(h, wd_ref[...], preferred_element_type=jnp.float32) + bd_ref[...], 0.0)
    rows = x.shape[0]
    aid = aid_ref[...]
    sid = sid_ref[...]
    na = aemb_ref.shape[0]
    ns = semb_ref.shape[0]
    aoh = (aid == lax.broadcasted_iota(jnp.int32, (rows, na), 1)).astype(jnp.float32)
    soh = (sid == lax.broadcasted_iota(jnp.int32, (rows, ns), 1)).astype(jnp.float32)
    a_e = jnp.dot(aoh, aemb_ref[...], preferred_element_type=jnp.float32)
    s_e = jnp.dot(soh, semb_ref[...], preferred_element_type=jnp.float32)
    hn = _ln_rows(h + a_e + s_e, lnng_ref[...], lnnb_ref[...])
    h_ref[...] = jnp.concatenate(
        [hn, jnp.zeros((rows, HD - NODE_DIM), jnp.float32)], axis=1
    ).astype(jnp.bfloat16)


def _node_init(x, atom_ids, aa_ids, atom_emb, aa_emb,
               ln_c_g, ln_c_b, W_p1, b_p1, W_p2, b_p2, W_d, b_d, ln_n_g, ln_n_b):
    nb = 10
    rows = N // nb
    full = lambda s: pl.BlockSpec(s, lambda b: (0, 0))
    return pl.pallas_call(
        _node_body,
        grid=(nb,),
        in_specs=[
            pl.BlockSpec((rows, 3), lambda b: (b, 0)),
            pl.BlockSpec((rows, 1), lambda b: (b, 0)),
            pl.BlockSpec((rows, 1), lambda b: (b, 0)),
            full(atom_emb.shape), full(aa_emb.shape),
            full((1, 3)), full((1, 3)),
            full(W_p1.shape), full((1, 12)),
            full(W_p2.shape), full((1, 48)),
            full(W_d.shape), full((1, 48)),
            full((1, 48)), full((1, 48)),
        ],
        out_specs=pl.BlockSpec((rows, HD), lambda b: (b, 0)),
        out_shape=jax.ShapeDtypeStruct((N, HD), jnp.bfloat16),
    )(x, atom_ids.reshape(N, 1).astype(jnp.int32), aa_ids.reshape(N, 1).astype(jnp.int32),
      atom_emb, aa_emb,
      ln_c_g.reshape(1, 3), ln_c_b.reshape(1, 3),
      W_p1, b_p1.reshape(1, 12), W_p2, b_p2.reshape(1, 48),
      W_d, b_d.reshape(1, 48), ln_n_g.reshape(1, 48), ln_n_b.reshape(1, 48))


def _sc_gather(h, idx0, idx1):
    mesh = plsc.VectorSubcoreMesh(core_axis_name="c", subcore_axis_name="s")

    @functools.partial(
        pl.kernel,
        out_type=[jax.ShapeDtypeStruct((EPAD, HD), jnp.bfloat16),
                  jax.ShapeDtypeStruct((EPAD, HD), jnp.bfloat16)],
        mesh=mesh,
        scratch_types=[
            pltpu.VMEM((CPW, CHUNK), jnp.int32),
            pltpu.VMEM((CPW, CHUNK), jnp.int32),
            pltpu.VMEM((CHUNK, HD), jnp.bfloat16),
            pltpu.VMEM((CHUNK, HD), jnp.bfloat16),
            pltpu.VMEM((CHUNK, HD), jnp.bfloat16),
            pltpu.VMEM((CHUNK, HD), jnp.bfloat16),
            pltpu.VMEM_SHARED((N, HD), jnp.bfloat16),
            pltpu.SemaphoreType.DMA,
            pltpu.SemaphoreType.DMA,
            pltpu.SemaphoreType.DMA,
            pltpu.SemaphoreType.DMA,
        ],
        compiler_params=pltpu.CompilerParams(use_tc_tiling_on_sc=False),
    )
    def k(h_hbm, i0_hbm, i1_hbm, hr_hbm, hc_hbm, i0v, i1v,
          ar, ac, br, bc, hs, sar, sac, sbr, sbc):
        wid = lax.axis_index("s") * 2 + lax.axis_index("c")
        cbase = wid * CPW
        @pl.when(lax.axis_index("s") == 0)
        def _():
            pltpu.sync_copy(h_hbm, hs)

        pltpu.sync_copy(i0_hbm.at[wid], i0v)
        pltpu.sync_copy(i1_hbm.at[wid], i1v)
        plsc.subcore_barrier()

        def gather(j, bufr, bufc, semr, semc):
            pltpu.async_copy(hs.at[i0v.at[j]], bufr, semr)
            pltpu.async_copy(hs.at[i1v.at[j]], bufc, semc)

        def wait(bufr, bufc, semr, semc):
            pltpu.make_async_copy(hs.at[i0v.at[0]], bufr, semr).wait()
            pltpu.make_async_copy(hs.at[i1v.at[0]], bufc, semc).wait()

        def write(j, bufr, bufc):
            dst = pl.ds((cbase + j) * CHUNK, CHUNK)
            pltpu.sync_copy(bufr, hr_hbm.at[dst, :])
            pltpu.sync_copy(bufc, hc_hbm.at[dst, :])

        gather(0, ar, ac, sar, sac)

        def body(i2, carry):
            j = i2 * 2
            gather(j + 1, br, bc, sbr, sbc)
            wait(ar, ac, sar, sac)
            write(j, ar, ac)
            gather(j + 2, ar, ac, sar, sac)
            wait(br, bc, sbr, sbc)
            write(j + 1, br, bc)
            return carry

        lax.fori_loop(0, (CPW - 2) // 2, body, 0, unroll=False)
        j = CPW - 2
        gather(j + 1, br, bc, sbr, sbc)
        wait(ar, ac, sar, sac)
        write(j, ar, ac)
        wait(br, bc, sbr, sbc)
        write(j + 1, br, bc)

    return k(h, idx0, idx1)


def _edge_attr_blk(hr, hc, we, be, lneg, lneb):
    her = jnp.maximum(jnp.dot(hr, we, preferred_element_type=jnp.float32) + be, 0.0)
    hec = jnp.maximum(jnp.dot(hc, we, preferred_element_type=jnp.float32) + be, 0.0)
    return _ln_rows((her + hec) * 0.5, lneg, lneb)


def _rowids(b, axis):
    if axis == 0:
        return lax.broadcasted_iota(jnp.int32, (BLK, 16), 0) + b * BLK
    return lax.broadcasted_iota(jnp.int32, (16, BLK), 1) + b * BLK


def _gid(rid):
    g = jnp.where(rid < E0, rid // EPG, (rid - E0) // NPG)
    return jnp.where(rid < ET, g, -1)


def _pass1_body(hr_ref, hc_ref, we_ref, be_ref, lneg_ref, lneb_ref,
                wg_ref, bg_ref, psum_ref):
    b = pl.program_id(0)
    ea = _edge_attr_blk(hr_ref[:, :NODE_DIM], hc_ref[:, :NODE_DIM],
                        we_ref[...], be_ref[...],
                        lneg_ref[...], lneb_ref[...])
    eg = jnp.maximum(jnp.dot(ea, wg_ref[...], preferred_element_type=jnp.float32) + bg_ref[...], 0.0)
    ohT = (_gid(_rowids(b, 1)) == lax.broadcasted_iota(jnp.int32, (16, BLK), 0)
           ).astype(jnp.float32)
    part = jnp.dot(ohT, eg, preferred_element_type=jnp.float32)

    @pl.when(b == 0)
    def _():
        psum_ref[...] = jnp.zeros_like(psum_ref)

    psum_ref[...] += part


def _pass1(hr, hc, W_e128, b_e, ln_e_g, ln_e_b, W_g, b_g):
    full = lambda s: pl.BlockSpec(s, lambda b: (0, 0))
    return pl.pallas_call(
        _pass1_body,
        grid=(NBLK,),
        in_specs=[
            pl.BlockSpec((BLK, HD), lambda b: (b, 0)),
            pl.BlockSpec((BLK, HD), lambda b: (b, 0)),
            full((NODE_DIM, EDGE_DIM)), full((1, EDGE_DIM)),
            full((1, EDGE_DIM)), full((1, EDGE_DIM)),
            full(W_g.shape), full((1, GLOBAL_DIM)),
        ],
        out_specs=pl.BlockSpec((16, GLOBAL_DIM), lambda b: (0, 0)),
        out_shape=jax.ShapeDtypeStruct((16, GLOBAL_DIM), jnp.float32),
    )(hr, hc, W_e128, b_e.reshape(1, EDGE_DIM), ln_e_g.reshape(1, EDGE_DIM),
      ln_e_b.reshape(1, EDGE_DIM), W_g, b_g.reshape(1, GLOBAL_DIM))


def _pass2_body(hr_ref, hc_ref, ps_ref, we_ref, be_ref, lneg_ref, lneb_ref,
                lnug_ref, lnub_ref, wa_ref, wb_ref, wc_ref, wd_ref, bm1_ref,
                wm2_ref, bm2_ref, lnog_ref, lnob_ref, out_ref):
    b = pl.program_id(0)
    u = _ln_rows(ps_ref[...] * (1.0 / CNT), lnug_ref[...], lnub_ref[...])
    oh = (_gid(_rowids(b, 0)) == lax.broadcasted_iota(jnp.int32, (BLK, 16), 1)
          ).astype(jnp.float32)
    usel = jnp.dot(oh, u, preferred_element_type=jnp.float32)

    hr = hr_ref[:, :NODE_DIM]
    hc = hc_ref[:, :NODE_DIM]
    ea = _edge_attr_blk(hr, hc, we_ref[...], be_ref[...], lneg_ref[...], lneb_ref[...])
    t = (jnp.dot(hr, wa_ref[...], preferred_element_type=jnp.float32)
         + jnp.dot(hc, wb_ref[...], preferred_element_type=jnp.float32)
         + jnp.dot(ea.astype(jnp.bfloat16), wc_ref[...], preferred_element_type=jnp.float32)
         + jnp.dot(usel.astype(jnp.bfloat16), wd_ref[...], preferred_element_type=jnp.float32)
         + bm1_ref[...])
    t = jnp.maximum(t, 0.0).astype(jnp.bfloat16)
    o = jnp.dot(t, wm2_ref[...], preferred_element_type=jnp.float32) + bm2_ref[...] + ea
    out_ref[...] = _ln_rows(o, lnog_ref[...], lnob_ref[...]).T


def _pass2(hr, hc, psums, W_e128, b_e, ln_e_g, ln_e_b, ln_u_g, ln_u_b,
           wa, wb, wc, wd, b_m1, W_m2, b_m2, ln_o_g, ln_o_b):
    full = lambda s: pl.BlockSpec(s, lambda b: (0, 0))
    h1 = wa.shape[1]
    return pl.pallas_call(
        _pass2_body,
        grid=(NBLK,),
        in_specs=[
            pl.BlockSpec((BLK, HD), lambda b: (b, 0)),
            pl.BlockSpec((BLK, HD), lambda b: (b, 0)),
            full((16, GLOBAL_DIM)),
            full((NODE_DIM, EDGE_DIM)), full((1, EDGE_DIM)),
            full((1, EDGE_DIM)), full((1, EDGE_DIM)),
            full((1, GLOBAL_DIM)), full((1, GLOBAL_DIM)),
            full((NODE_DIM, h1)), full((NODE_DIM, h1)),
            full((EDGE_DIM, h1)), full((GLOBAL_DIM, h1)),
            full((1, h1)),
            full(W_m2.shape), full((1, EDGE_DIM)),
            full((1, EDGE_DIM)), full((1, EDGE_DIM)),
        ],
        out_specs=pl.BlockSpec((EDGE_DIM, BLK), lambda b: (0, b)),
        out_shape=jax.ShapeDtypeStruct((EDGE_DIM, ET), jnp.float32),
    )(hr, hc, psums, W_e128, b_e.reshape(1, EDGE_DIM), ln_e_g.reshape(1, EDGE_DIM),
      ln_e_b.reshape(1, EDGE_DIM), ln_u_g.reshape(1, GLOBAL_DIM),
      ln_u_b.reshape(1, GLOBAL_DIM), wa, wb, wc, wd, b_m1.reshape(1, h1),
      W_m2, b_m2.reshape(1, EDGE_DIM), ln_o_g.reshape(1, EDGE_DIM),
      ln_o_b.reshape(1, EDGE_DIM))


def kernel(x, atom_ids, aa_ids, edge_index, ln_c_g, ln_c_b, W_p1, b_p1, W_p2, b_p2,
           W_d, b_d, atom_emb, aa_emb, ln_n_g, ln_n_b, W_e, b_e, ln_e_g, ln_e_b,
           W_g, b_g, ln_u_g, ln_u_b, W_m1, b_m1, W_m2, b_m2, ln_o_g, ln_o_b):
    h = _node_init(x, atom_ids, aa_ids, atom_emb, aa_emb,
                   ln_c_g, ln_c_b, W_p1, b_p1, W_p2, b_p2, W_d, b_d, ln_n_g, ln_n_b)

    loops = jnp.arange(N, dtype=jnp.int32)
    pad = jnp.zeros((EPAD - ET,), jnp.int32)
    ei0 = jnp.concatenate([edge_index[0].astype(jnp.int32), loops, pad]).reshape(NW, CPW, CHUNK)
    ei1 = jnp.concatenate([edge_index[1].astype(jnp.int32), loops, pad]).reshape(NW, CPW, CHUNK)

    hr, hc = _sc_gather(h, ei0, ei1)

    bf = jnp.bfloat16
    W_e16 = W_e.astype(bf)
    wa = W_m1[:NODE_DIM].astype(bf)
    wb = W_m1[NODE_DIM:2 * NODE_DIM].astype(bf)
    wc = W_m1[2 * NODE_DIM:2 * NODE_DIM + EDGE_DIM].astype(bf)
    wd = W_m1[2 * NODE_DIM + EDGE_DIM:].astype(bf)

    psums = _pass1(hr, hc, W_e16, b_e, ln_e_g, ln_e_b, W_g, b_g)
    out_t = _pass2(hr, hc, psums, W_e16, b_e, ln_e_g, ln_e_b, ln_u_g, ln_u_b,
                   wa, wb, wc, wd, b_m1, W_m2.astype(bf), b_m2, ln_o_g, ln_o_b)
    return out_t.T


# speedup vs baseline: 3.6575x; 1.7382x over previous
import functools

import jax
import jax.numpy as jnp
from jax import lax
from jax.experimental import pallas as pl
from jax.experimental.pallas import tpu as pltpu
from jax.experimental.pallas import tpu_sc as plsc

N = 10000
E0 = 320000
ET = E0 + N
NODE_DIM = 48
HD = 128
EDGE_DIM = 32
GLOBAL_DIM = 32
EPG = 32000
NPG = 1000
CNT = float(EPG + NPG)

NW = 32
CHUNK = 128
CPW = 88
NCH = NW * CPW
EPAD = NCH * CHUNK

BLK = 3072
NBLK = -(-ET // BLK)

_EPS = 1e-5


def _ln_rows(v, g, b):
    mu = jnp.mean(v, axis=-1, keepdims=True)
    var = jnp.mean(v * v, axis=-1, keepdims=True) - mu * mu
    return (v - mu) * jax.lax.rsqrt(var + _EPS) * g + b


def _node_body(x_ref, aid_ref, sid_ref, aemb_ref, semb_ref,
               lncg_ref, lncb_ref, wp1_ref, bp1_ref, wp2_ref, bp2_ref,
               wd_ref, bd_ref, lnng_ref, lnnb_ref, h_ref):
    x = x_ref[...]
    h = _ln_rows(x, lncg_ref[...], lncb_ref[...])
    h = jnp.maximum(jnp.dot(h, wp1_ref[...], preferred_element_type=jnp.float32) + bp1_ref[...], 0.0)
    h = jnp.maximum(jnp.dot(h, wp2_ref[...], preferred_element_type=jnp.float32) + bp2_ref[...], 0.0)
    h = jnp.maximum(jnp.dot(h, wd_ref[...], preferred_element_type=jnp.float32) + bd_ref[...], 0.0)
    rows = x.shape[0]
    aid = aid_ref[...]
    sid = sid_ref[...]
    na = aemb_ref.shape[0]
    ns = semb_ref.shape[0]
    aoh = (aid == lax.broadcasted_iota(jnp.int32, (rows, na), 1)).astype(jnp.float32)
    soh = (sid == lax.broadcasted_iota(jnp.int32, (rows, ns), 1)).astype(jnp.float32)
    a_e = jnp.dot(aoh, aemb_ref[...], preferred_element_type=jnp.float32)
    s_e = jnp.dot(soh, semb_ref[...], preferred_element_type=jnp.float32)
    h_ref[...] = _ln_rows(h + a_e + s_e, lnng_ref[...], lnnb_ref[...])


def _node_init(x, atom_ids, aa_ids, atom_emb, aa_emb,
               ln_c_g, ln_c_b, W_p1, b_p1, W_p2, b_p2, W_d, b_d, ln_n_g, ln_n_b):
    nb = 10
    rows = N // nb
    full = lambda s: pl.BlockSpec(s, lambda b: (0, 0))
    return pl.pallas_call(
        _node_body,
        grid=(nb,),
        in_specs=[
            pl.BlockSpec((rows, 3), lambda b: (b, 0)),
            pl.BlockSpec((rows, 1), lambda b: (b, 0)),
            pl.BlockSpec((rows, 1), lambda b: (b, 0)),
            full(atom_emb.shape), full(aa_emb.shape),
            full((1, 3)), full((1, 3)),
            full(W_p1.shape), full((1, 12)),
            full(W_p2.shape), full((1, 48)),
            full(W_d.shape), full((1, 48)),
            full((1, 48)), full((1, 48)),
        ],
        out_specs=pl.BlockSpec((rows, NODE_DIM), lambda b: (b, 0)),
        out_shape=jax.ShapeDtypeStruct((N, NODE_DIM), jnp.float32),
    )(x, atom_ids.reshape(N, 1).astype(jnp.int32), aa_ids.reshape(N, 1).astype(jnp.int32),
      atom_emb, aa_emb,
      ln_c_g.reshape(1, 3), ln_c_b.reshape(1, 3),
      W_p1, b_p1.reshape(1, 12), W_p2, b_p2.reshape(1, 48),
      W_d, b_d.reshape(1, 48), ln_n_g.reshape(1, 48), ln_n_b.reshape(1, 48))


def _sc_gather(h, idx0, idx1):
    mesh = plsc.VectorSubcoreMesh(core_axis_name="c", subcore_axis_name="s")

    @functools.partial(
        pl.kernel,
        out_type=[jax.ShapeDtypeStruct((EPAD, HD), jnp.float32),
                  jax.ShapeDtypeStruct((EPAD, HD), jnp.float32)],
        mesh=mesh,
        scratch_types=[
            pltpu.VMEM((CPW, CHUNK), jnp.int32),
            pltpu.VMEM((CPW, CHUNK), jnp.int32),
            pltpu.VMEM((CHUNK, NODE_DIM), jnp.float32),
            pltpu.VMEM((CHUNK, NODE_DIM), jnp.float32),
            pltpu.VMEM((CHUNK, NODE_DIM), jnp.float32),
            pltpu.VMEM((CHUNK, NODE_DIM), jnp.float32),
            pltpu.VMEM_SHARED((N, NODE_DIM), jnp.float32),
            pltpu.SemaphoreType.DMA,
            pltpu.SemaphoreType.DMA,
            pltpu.SemaphoreType.DMA,
            pltpu.SemaphoreType.DMA,
        ],
        compiler_params=pltpu.CompilerParams(use_tc_tiling_on_sc=False),
    )
    def k(h_hbm, i0_hbm, i1_hbm, hr_hbm, hc_hbm, i0v, i1v,
          ar, ac, br, bc, hs, sar, sac, sbr, sbc):
        wid = lax.axis_index("s") * 2 + lax.axis_index("c")
        cbase = wid * CPW
        @pl.when(lax.axis_index("s") == 0)
        def _():
            pltpu.sync_copy(h_hbm, hs)

        pltpu.sync_copy(i0_hbm.at[wid], i0v)
        pltpu.sync_copy(i1_hbm.at[wid], i1v)
        plsc.subcore_barrier()

        def gather(j, bufr, bufc, semr, semc):
            pltpu.async_copy(hs.at[i0v.at[j]], bufr, semr)
            pltpu.async_copy(hs.at[i1v.at[j]], bufc, semc)

        def wait(bufr, bufc, semr, semc):
            pltpu.make_async_copy(hs.at[i0v.at[0]], bufr, semr).wait()
            pltpu.make_async_copy(hs.at[i1v.at[0]], bufc, semc).wait()

        def write(j, bufr, bufc):
            dst = pl.ds((cbase + j) * CHUNK, CHUNK)
            pltpu.sync_copy(bufr, hr_hbm.at[dst, pl.ds(0, NODE_DIM)])
            pltpu.sync_copy(bufc, hc_hbm.at[dst, pl.ds(0, NODE_DIM)])

        gather(0, ar, ac, sar, sac)

        def body(i2, carry):
            j = i2 * 2
            gather(j + 1, br, bc, sbr, sbc)
            wait(ar, ac, sar, sac)
            write(j, ar, ac)
            gather(j + 2, ar, ac, sar, sac)
            wait(br, bc, sbr, sbc)
            write(j + 1, br, bc)
            return carry

        lax.fori_loop(0, (CPW - 2) // 2, body, 0, unroll=False)
        j = CPW - 2
        gather(j + 1, br, bc, sbr, sbc)
        wait(ar, ac, sar, sac)
        write(j, ar, ac)
        wait(br, bc, sbr, sbc)
        write(j + 1, br, bc)

    return k(h, idx0, idx1)


def _edge_attr_blk(hr, hc, we, be, lneg, lneb):
    her = jnp.maximum(jnp.dot(hr, we, preferred_element_type=jnp.float32) + be, 0.0)
    hec = jnp.maximum(jnp.dot(hc, we, preferred_element_type=jnp.float32) + be, 0.0)
    return _ln_rows((her + hec) * 0.5, lneg, lneb)


def _rowids(b, axis):
    if axis == 0:
        return lax.broadcasted_iota(jnp.int32, (BLK, 16), 0) + b * BLK
    return lax.broadcasted_iota(jnp.int32, (16, BLK), 1) + b * BLK


def _gid(rid):
    g = jnp.where(rid < E0, rid // EPG, (rid - E0) // NPG)
    return jnp.where(rid < ET, g, -1)


def _pass1_body(hr_ref, hc_ref, we_ref, be_ref, lneg_ref, lneb_ref,
                wg_ref, bg_ref, psum_ref):
    b = pl.program_id(0)
    ea = _edge_attr_blk(hr_ref[:, :NODE_DIM], hc_ref[:, :NODE_DIM],
                        we_ref[...], be_ref[...],
                        lneg_ref[...], lneb_ref[...])
    eg = jnp.maximum(jnp.dot(ea, wg_ref[...], preferred_element_type=jnp.float32) + bg_ref[...], 0.0)
    ohT = (_gid(_rowids(b, 1)) == lax.broadcasted_iota(jnp.int32, (16, BLK), 0)
           ).astype(jnp.float32)
    part = jnp.dot(ohT, eg, preferred_element_type=jnp.float32)

    @pl.when(b == 0)
    def _():
        psum_ref[...] = jnp.zeros_like(psum_ref)

    psum_ref[...] += part


def _pass1(hr, hc, W_e128, b_e, ln_e_g, ln_e_b, W_g, b_g):
    full = lambda s: pl.BlockSpec(s, lambda b: (0, 0))
    return pl.pallas_call(
        _pass1_body,
        grid=(NBLK,),
        in_specs=[
            pl.BlockSpec((BLK, HD), lambda b: (b, 0)),
            pl.BlockSpec((BLK, HD), lambda b: (b, 0)),
            full((NODE_DIM, EDGE_DIM)), full((1, EDGE_DIM)),
            full((1, EDGE_DIM)), full((1, EDGE_DIM)),
            full(W_g.shape), full((1, GLOBAL_DIM)),
        ],
        out_specs=pl.BlockSpec((16, GLOBAL_DIM), lambda b: (0, 0)),
        out_shape=jax.ShapeDtypeStruct((16, GLOBAL_DIM), jnp.float32),
    )(hr, hc, W_e128, b_e.reshape(1, EDGE_DIM), ln_e_g.reshape(1, EDGE_DIM),
      ln_e_b.reshape(1, EDGE_DIM), W_g, b_g.reshape(1, GLOBAL_DIM))


def _pass2_body(hr_ref, hc_ref, ps_ref, we_ref, be_ref, lneg_ref, lneb_ref,
                lnug_ref, lnub_ref, wa_ref, wb_ref, wc_ref, wd_ref, bm1_ref,
                wm2_ref, bm2_ref, lnog_ref, lnob_ref, out_ref):
    b = pl.program_id(0)
    u = _ln_rows(ps_ref[...] * (1.0 / CNT), lnug_ref[...], lnub_ref[...])
    oh = (_gid(_rowids(b, 0)) == lax.broadcasted_iota(jnp.int32, (BLK, 16), 1)
          ).astype(jnp.float32)
    usel = jnp.dot(oh, u, preferred_element_type=jnp.float32)

    hr = hr_ref[:, :NODE_DIM]
    hc = hc_ref[:, :NODE_DIM]
    ea = _edge_attr_blk(hr, hc, we_ref[...], be_ref[...], lneg_ref[...], lneb_ref[...])
    t = (jnp.dot(hr, wa_ref[...], preferred_element_type=jnp.float32)
         + jnp.dot(hc, wb_ref[...], preferred_element_type=jnp.float32)
         + jnp.dot(ea, wc_ref[...], preferred_element_type=jnp.float32)
         + jnp.dot(usel, wd_ref[...], preferred_element_type=jnp.float32)
         + bm1_ref[...])
    t = jnp.maximum(t, 0.0)
    o = jnp.dot(t, wm2_ref[...], preferred_element_type=jnp.float32) + bm2_ref[...] + ea
    out_ref[...] = _ln_rows(o, lnog_ref[...], lnob_ref[...]).T


def _pass2(hr, hc, psums, W_e128, b_e, ln_e_g, ln_e_b, ln_u_g, ln_u_b,
           wa, wb, wc, wd, b_m1, W_m2, b_m2, ln_o_g, ln_o_b):
    full = lambda s: pl.BlockSpec(s, lambda b: (0, 0))
    h1 = wa.shape[1]
    return pl.pallas_call(
        _pass2_body,
        grid=(NBLK,),
        in_specs=[
            pl.BlockSpec((BLK, HD), lambda b: (b, 0)),
            pl.BlockSpec((BLK, HD), lambda b: (b, 0)),
            full((16, GLOBAL_DIM)),
            full((NODE_DIM, EDGE_DIM)), full((1, EDGE_DIM)),
            full((1, EDGE_DIM)), full((1, EDGE_DIM)),
            full((1, GLOBAL_DIM)), full((1, GLOBAL_DIM)),
            full((NODE_DIM, h1)), full((NODE_DIM, h1)),
            full((EDGE_DIM, h1)), full((GLOBAL_DIM, h1)),
            full((1, h1)),
            full(W_m2.shape), full((1, EDGE_DIM)),
            full((1, EDGE_DIM)), full((1, EDGE_DIM)),
        ],
        out_specs=pl.BlockSpec((EDGE_DIM, BLK), lambda b: (0, b)),
        out_shape=jax.ShapeDtypeStruct((EDGE_DIM, ET), jnp.float32),
    )(hr, hc, psums, W_e128, b_e.reshape(1, EDGE_DIM), ln_e_g.reshape(1, EDGE_DIM),
      ln_e_b.reshape(1, EDGE_DIM), ln_u_g.reshape(1, GLOBAL_DIM),
      ln_u_b.reshape(1, GLOBAL_DIM), wa, wb, wc, wd, b_m1.reshape(1, h1),
      W_m2, b_m2.reshape(1, EDGE_DIM), ln_o_g.reshape(1, EDGE_DIM),
      ln_o_b.reshape(1, EDGE_DIM))


def kernel(x, atom_ids, aa_ids, edge_index, ln_c_g, ln_c_b, W_p1, b_p1, W_p2, b_p2,
           W_d, b_d, atom_emb, aa_emb, ln_n_g, ln_n_b, W_e, b_e, ln_e_g, ln_e_b,
           W_g, b_g, ln_u_g, ln_u_b, W_m1, b_m1, W_m2, b_m2, ln_o_g, ln_o_b):
    h = _node_init(x, atom_ids, aa_ids, atom_emb, aa_emb,
                   ln_c_g, ln_c_b, W_p1, b_p1, W_p2, b_p2, W_d, b_d, ln_n_g, ln_n_b)

    loops = jnp.arange(N, dtype=jnp.int32)
    pad = jnp.zeros((EPAD - ET,), jnp.int32)
    ei0 = jnp.concatenate([edge_index[0].astype(jnp.int32), loops, pad]).reshape(NW, CPW, CHUNK)
    ei1 = jnp.concatenate([edge_index[1].astype(jnp.int32), loops, pad]).reshape(NW, CPW, CHUNK)

    hr, hc = _sc_gather(h, ei0, ei1)

    wa = W_m1[:NODE_DIM]
    wb = W_m1[NODE_DIM:2 * NODE_DIM]
    wc = W_m1[2 * NODE_DIM:2 * NODE_DIM + EDGE_DIM]
    wd = W_m1[2 * NODE_DIM + EDGE_DIM:]

    psums = _pass1(hr, hc, W_e, b_e, ln_e_g, ln_e_b, W_g, b_g)
    out_t = _pass2(hr, hc, psums, W_e, b_e, ln_e_g, ln_e_b, ln_u_g, ln_u_b,
                   wa, wb, wc, wd, b_m1, W_m2, b_m2, ln_o_g, ln_o_b)
    return out_t.T
